# baseline (speedup 1.0000x reference)
import jax
import jax.numpy as jnp
from jax.experimental import pallas as pl
from jax.experimental.pallas import tpu as pltpu

_SCALE = 1.0 / 32
_P = 7


def _roi_pool_kernel(bidx_ref, valid_ref, x0_ref, x1_ref, wk_ref, hk_ref,
                     feat_ref, cmask_ref, rmask_ref, out_ref, s1_ref):
    s = pl.program_id(0)

    @pl.when(s == 0)
    def _init():
        s1_ref[...] = jnp.full_like(s1_ref, -jnp.inf)

    for rr in range(4):
        r = s * 4 + rr
        b = bidx_ref[r]
        cm = cmask_ref[r]
        rm = rmask_ref[r]
        v = valid_ref[r]
        h0 = x0_ref[r]
        h1 = x1_ref[r]
        s1r = s1_ref.at[rr]
        k8s = [wk_ref[r, jj] * 8 for jj in range(_P)]

        def _chunk(kb, carry):
            base = jnp.minimum(h0 + kb * 8, 56)
            for j in range(_P):
                cmj = cm[j][:, None]
                for hh in range(8):
                    slab = feat_ref[b, base + hh, pl.ds(k8s[j], 24), :]
                    s1r[j, base + hh] = jnp.max(slab + cmj, axis=0)
            return carry

        jax.lax.fori_loop(0, (h1 - h0 + 7) // 8, _chunk, 0)

        for i in range(_P):
            hk8 = hk_ref[r, i] * 8
            rmi = rm[i][:, None]
            for j in range(_P):
                val = jnp.max(s1r[j, pl.ds(hk8, 24)] + rmi, axis=0)
                out_ref[rr, i, j] = jnp.where(v > 0, val, 0.0)


def kernel(features, rois):
    N, C, H, W = features.shape
    R = rois.shape[0]
    P = _P

    bidx = rois[:, 0].astype(jnp.int32)
    bbox = jnp.round(rois[:, 1:] * _SCALE).astype(jnp.int32)
    x0 = jnp.clip(bbox[:, 0], 0, W - 1)
    y0 = jnp.clip(bbox[:, 1], 0, H - 1)
    x1 = jnp.clip(bbox[:, 2], 0, W - 1)
    y1 = jnp.clip(bbox[:, 3], 0, H - 1)
    valid = ((x0 < x1) & (y0 < y1)).astype(jnp.int32)
    Lh = x1 - x0
    Lw = y1 - y0

    j = jnp.arange(P)
    neg = jnp.float32(-jnp.inf)
    ws = y0[:, None] + (j[None, :] * Lw[:, None]) // P
    we = y0[:, None] + ((j[None, :] + 1) * Lw[:, None] + P - 1) // P
    wk = jnp.minimum(ws // 8, (W - 24) // 8)
    wglob = 8 * wk[:, :, None] + jnp.arange(24)[None, None, :]
    cmask = jnp.where(
        (wglob >= ws[:, :, None]) & (wglob < we[:, :, None]),
        jnp.float32(0), neg,
    )
    hs = x0[:, None] + (j[None, :] * Lh[:, None]) // P
    he = x0[:, None] + ((j[None, :] + 1) * Lh[:, None] + P - 1) // P
    hk = jnp.minimum(hs // 8, (H - 24) // 8)
    hglob = 8 * hk[:, :, None] + jnp.arange(24)[None, None, :]
    rmask = jnp.where(
        (hglob >= hs[:, :, None]) & (hglob < he[:, :, None]),
        jnp.float32(0), neg,
    )

    fhwc = jnp.transpose(features, (0, 2, 3, 1))

    out = pl.pallas_call(
        _roi_pool_kernel,
        grid_spec=pltpu.PrefetchScalarGridSpec(
            num_scalar_prefetch=6,
            grid=(R // 4,),
            in_specs=[
                pl.BlockSpec((N, H, W, C), lambda r, *_: (0, 0, 0, 0)),
                pl.BlockSpec((R, P, 24), lambda r, *_: (0, 0, 0)),
                pl.BlockSpec((R, P, 24), lambda r, *_: (0, 0, 0)),
            ],
            out_specs=pl.BlockSpec((4, P, P, C), lambda r, *_: (r, 0, 0, 0)),
            scratch_shapes=[pltpu.VMEM((4, P, H, C), jnp.float32)],
        ),
        out_shape=jax.ShapeDtypeStruct((R, P, P, C), jnp.float32),
        compiler_params=pltpu.CompilerParams(
            dimension_semantics=("arbitrary",),
            vmem_limit_bytes=60 * 1024 * 1024,
        ),
        name="roi_pool",
    )(bidx, valid, x0, x1, wk, hk, fhwc, cmask, rmask)

    return jnp.transpose(out, (0, 3, 1, 2))
